# baseline (speedup 1.0000x reference)
import dataclasses
import functools

import jax
import jax.numpy as jnp
from jax import lax
from jax.experimental import pallas as pl
from jax.experimental.pallas import tpu as pltpu
from jax.experimental.pallas import tpu_sc as plsc

N = 10000
E = 320000
G = 64
D = 128
D_OUT = 40

NC = 2
NS = 16
C = 64
CH = 160
W = 16
NW = CH // W
EPT = CH * C
EPAD = NC * NS * EPT
ACC_ROWS = 10240
ROWS_PER_SUB = ACC_ROWS // NS

BLK = 1000


def _sc_aggregate(x, src, dst, want_counts):
    mesh = plsc.VectorSubcoreMesh(core_axis_name="c", subcore_axis_name="s")
    out_type = [jax.ShapeDtypeStruct((NC, ACC_ROWS, D), jnp.float32)]
    scratch = [
        pltpu.VMEM((2, W, C), jnp.int32),
        pltpu.VMEM((2, W, C), jnp.int32),
        pltpu.VMEM((C, D), jnp.float32),
        pltpu.VMEM((C, D), jnp.float32),
        pltpu.VMEM_SHARED((ACC_ROWS, D), jnp.float32),
        pltpu.SemaphoreType.DMA,
        pltpu.SemaphoreType.DMA,
        pltpu.SemaphoreType.DMA,
    ]
    if want_counts:
        out_type.append(jax.ShapeDtypeStruct((NC * NS, ACC_ROWS), jnp.float32))
        scratch.append(pltpu.VMEM((ACC_ROWS,), jnp.float32))

    def body(x_hbm, src_hbm, dst_hbm, *rest):
        if want_counts:
            (p_hbm, cnt_hbm, srcb, dstb, rows0, rows1, acc,
             sem0, sem1, semi, cntloc) = rest
        else:
            (p_hbm, srcb, dstb, rows0, rows1, acc,
             sem0, sem1, semi) = rest
        c = lax.axis_index("c")
        s = lax.axis_index("s")
        tile = c * NS + s
        base = tile * CH

        z16 = jnp.zeros((16,), jnp.float32)

        pltpu.async_copy(src_hbm.at[pl.ds(base, W)], srcb.at[0], semi)
        pltpu.async_copy(dst_hbm.at[pl.ds(base, W)], dstb.at[0], semi)

        @pl.loop(0, C)
        def _(r):
            @pl.loop(0, D // 16)
            def _(j):
                rows1[r, pl.ds(j * 16, 16)] = z16

        if want_counts:
            @pl.loop(0, ACC_ROWS // 16)
            def _(i):
                cntloc[pl.ds(i * 16, 16)] = z16

        @pl.loop(0, ROWS_PER_SUB // C)
        def _(k):
            pltpu.sync_copy(rows1, acc.at[pl.ds(s * ROWS_PER_SUB + k * C, C)])

        pltpu.make_async_copy(src_hbm.at[pl.ds(base, W)], srcb.at[0],
                              semi).wait()
        pltpu.make_async_copy(dst_hbm.at[pl.ds(base, W)], dstb.at[0],
                              semi).wait()
        pltpu.async_copy(src_hbm.at[pl.ds(base + W, W)], srcb.at[1], semi)
        pltpu.async_copy(dst_hbm.at[pl.ds(base + W, W)], dstb.at[1], semi)
        pltpu.async_copy(x_hbm.at[srcb.at[0, 0]], rows0, sem0)

        plsc.subcore_barrier()

        ones16 = jnp.full((16,), 1.0, jnp.float32)

        def hist(slot, a):
            if want_counts:
                @pl.loop(0, C // 16)
                def _(j):
                    idx16 = dstb[slot, a, pl.ds(j * 16, 16)]
                    plsc.addupdate_scatter(cntloc, [idx16], ones16)

        @pl.loop(0, NW)
        def _(w):
            slot = lax.rem(w, 2)

            @pl.loop(0, W // 2)
            def _(k):
                a = 2 * k
                last = k == W // 2 - 1
                pltpu.make_async_copy(x_hbm.at[srcb.at[slot, a]], rows0,
                                      sem0).wait()
                pltpu.async_copy(x_hbm.at[srcb.at[slot, a + 1]], rows1, sem1)
                hist(slot, a)
                pltpu.sync_copy(rows0, acc.at[dstb.at[slot, a]], add=True)
                pltpu.make_async_copy(x_hbm.at[srcb.at[slot, a + 1]], rows1,
                                      sem1).wait()

                @pl.when(jnp.logical_not(last))
                def _():
                    pltpu.async_copy(x_hbm.at[srcb.at[slot, a + 2]], rows0,
                                     sem0)

                @pl.when(jnp.logical_and(last, w < NW - 1))
                def _():
                    pltpu.make_async_copy(
                        src_hbm.at[pl.ds(base + (w + 1) * W, W)],
                        srcb.at[1 - slot], semi).wait()
                    pltpu.make_async_copy(
                        dst_hbm.at[pl.ds(base + (w + 1) * W, W)],
                        dstb.at[1 - slot], semi).wait()
                    pltpu.async_copy(x_hbm.at[srcb.at[1 - slot, 0]], rows0,
                                     sem0)

                hist(slot, a + 1)
                pltpu.sync_copy(rows1, acc.at[dstb.at[slot, a + 1]], add=True)

                @pl.when(jnp.logical_and(last, w < NW - 2))
                def _():
                    pltpu.async_copy(src_hbm.at[pl.ds(base + (w + 2) * W, W)],
                                     srcb.at[slot], semi)
                    pltpu.async_copy(dst_hbm.at[pl.ds(base + (w + 2) * W, W)],
                                     dstb.at[slot], semi)

        plsc.subcore_barrier()

        @pl.loop(0, ROWS_PER_SUB // C)
        def _(k):
            r0 = s * ROWS_PER_SUB + k * C
            pltpu.sync_copy(acc.at[pl.ds(r0, C)], p_hbm.at[c, pl.ds(r0, C)])
        if want_counts:
            pltpu.sync_copy(cntloc, cnt_hbm.at[tile])

    cp = pltpu.CompilerParams()
    if "needs_layout_passes" in pltpu.CompilerParams.__dataclass_fields__:
        cp = dataclasses.replace(cp, needs_layout_passes=False)
    fn = pl.kernel(body, mesh=mesh, out_type=tuple(out_type),
                   scratch_types=tuple(scratch), compiler_params=cp)
    return fn(x, src, dst)


def _invcnt_body(cnt_ref, o_ref):
    ones = jnp.ones((NC * NS, 1), jnp.float32)
    col = lax.dot_general(cnt_ref[...], ones, (((0,), (0,)), ((), ())),
                          preferred_element_type=jnp.float32)
    o_ref[...] = 1.0 / jnp.maximum(col, 1.0)


def _tc_invcnt(cnt):
    return pl.pallas_call(
        _invcnt_body,
        out_shape=jax.ShapeDtypeStruct((ACC_ROWS, 1), jnp.float32),
    )(cnt)


def _layer1_body(p_ref, inv_ref, x_ref, wl_ref, b_ref, wr_ref, o_ref):
    agg = (p_ref[0] + p_ref[1]) * inv_ref[...]
    h = (jnp.dot(agg, wl_ref[...], preferred_element_type=jnp.float32)
         + b_ref[...]
         + jnp.dot(x_ref[...], wr_ref[...], preferred_element_type=jnp.float32))
    o_ref[...] = jnp.maximum(h, 0.0)


def _tc_layer1(p, inv, x, Wl, b, Wr):
    return pl.pallas_call(
        _layer1_body,
        grid=(N // BLK,),
        in_specs=[
            pl.BlockSpec((NC, BLK, D), lambda i: (0, i, 0)),
            pl.BlockSpec((BLK, 1), lambda i: (i, 0)),
            pl.BlockSpec((BLK, D), lambda i: (i, 0)),
            pl.BlockSpec((D, D), lambda i: (0, 0)),
            pl.BlockSpec((1, D), lambda i: (0, 0)),
            pl.BlockSpec((D, D), lambda i: (0, 0)),
        ],
        out_specs=pl.BlockSpec((BLK, D), lambda i: (i, 0)),
        out_shape=jax.ShapeDtypeStruct((N, D), jnp.float32),
        compiler_params=pltpu.CompilerParams(
            dimension_semantics=("parallel",)),
    )(p, inv, x, Wl, b, Wr)


def _layer2_body(p_ref, inv_ref, h1_ref, wl_ref, b_ref, wr_ref, batch_ref,
                 wf_ref, bf_ref, o_ref, pool_acc, cntp_acc):
    i = pl.program_id(0)

    @pl.when(i == 0)
    def _():
        pool_acc[...] = jnp.zeros_like(pool_acc)
        cntp_acc[...] = jnp.zeros_like(cntp_acc)

    agg = (p_ref[0] + p_ref[1]) * inv_ref[...]
    h2 = (jnp.dot(agg, wl_ref[...], preferred_element_type=jnp.float32)
          + b_ref[...]
          + jnp.dot(h1_ref[...], wr_ref[...],
                    preferred_element_type=jnp.float32))
    h2 = jnp.maximum(h2, 0.0)

    ids = batch_ref[0, 0, :]
    mask = (lax.broadcasted_iota(jnp.int32, (G, BLK), 0)
            == ids[None, :]).astype(jnp.float32)
    pool_acc[...] += jnp.dot(mask, h2, preferred_element_type=jnp.float32)
    cntp_acc[...] += jnp.sum(mask, axis=1, keepdims=True)

    @pl.when(i == N // BLK - 1)
    def _():
        pooled = pool_acc[...] / jnp.maximum(cntp_acc[...], 1.0)
        logits = (jnp.dot(pooled, wf_ref[...],
                          preferred_element_type=jnp.float32) + bf_ref[...])
        m = jnp.max(logits, axis=-1, keepdims=True)
        lse = jnp.log(jnp.sum(jnp.exp(logits - m), axis=-1, keepdims=True)) + m
        o_ref[...] = logits - lse


def _tc_layer2(p, inv, h1, Wl, b, Wr, batch3, Wf, bf):
    return pl.pallas_call(
        _layer2_body,
        grid=(N // BLK,),
        in_specs=[
            pl.BlockSpec((NC, BLK, D), lambda i: (0, i, 0)),
            pl.BlockSpec((BLK, 1), lambda i: (i, 0)),
            pl.BlockSpec((BLK, D), lambda i: (i, 0)),
            pl.BlockSpec((D, D), lambda i: (0, 0)),
            pl.BlockSpec((1, D), lambda i: (0, 0)),
            pl.BlockSpec((D, D), lambda i: (0, 0)),
            pl.BlockSpec((1, 1, BLK), lambda i: (i, 0, 0)),
            pl.BlockSpec((D, D_OUT), lambda i: (0, 0)),
            pl.BlockSpec((1, D_OUT), lambda i: (0, 0)),
        ],
        out_specs=pl.BlockSpec((G, D_OUT), lambda i: (0, 0)),
        out_shape=jax.ShapeDtypeStruct((G, D_OUT), jnp.float32),
        scratch_shapes=[
            pltpu.VMEM((G, D), jnp.float32),
            pltpu.VMEM((G, D), jnp.float32),
        ],
        compiler_params=pltpu.CompilerParams(
            dimension_semantics=("arbitrary",)),
    )(p, inv, h1, Wl, b, Wr, batch3, Wf, bf)


def kernel(x, edge_index, batch, W1l, b1l, W1r, W2l, b2l, W2r, Wf, bf):
    src = edge_index[0].astype(jnp.int32)
    dst = edge_index[1].astype(jnp.int32)
    pad = EPAD - E
    src_p = jnp.concatenate([src, jnp.zeros((pad,), jnp.int32)])
    src_p = src_p.reshape(NC * NS * CH, C)
    dst_p = jnp.concatenate([dst, jnp.full((pad,), N, jnp.int32)])
    dst_p = dst_p.reshape(NC * NS * CH, C)
    batch3 = batch.astype(jnp.int32).reshape(N // BLK, 1, BLK)
    b1 = b1l.reshape(1, D)
    b2 = b2l.reshape(1, D)
    bfr = bf.reshape(1, D_OUT)

    p1, cnt = _sc_aggregate(x, src_p, dst_p, want_counts=True)
    inv = _tc_invcnt(cnt)
    h1 = _tc_layer1(p1, inv, x, W1l, b1, W1r)
    (p2,) = _sc_aggregate(h1, src_p, dst_p, want_counts=False)
    return _tc_layer2(p2, inv, h1, W2l, b2, W2r, batch3, Wf, bfr)


# speedup vs baseline: 2.5610x; 2.5610x over previous
import dataclasses
import functools

import jax
import jax.numpy as jnp
from jax import lax
from jax.experimental import pallas as pl
from jax.experimental.pallas import tpu as pltpu
from jax.experimental.pallas import tpu_sc as plsc

N = 10000
E = 320000
G = 64
D = 128
D_OUT = 40

NC = 2
NS = 16
C = 64
CH = 160
W = 16
NW = CH // W
EPT = CH * C
EPAD = NC * NS * EPT
ACC_ROWS = 10240
ROWS_PER_SUB = ACC_ROWS // NS

BLK = 1000


def _sc_aggregate(x, src, dst, want_counts):
    mesh = plsc.VectorSubcoreMesh(core_axis_name="c", subcore_axis_name="s")
    out_type = [jax.ShapeDtypeStruct((NC, ACC_ROWS, D), jnp.float32)]
    scratch = [
        pltpu.VMEM((2, W, C), jnp.int32),
        pltpu.VMEM((2, W, C), jnp.int32),
        pltpu.VMEM((C, D), jnp.float32),
        pltpu.VMEM((C, D), jnp.float32),
        pltpu.VMEM_SHARED((ACC_ROWS, D), jnp.float32),
        pltpu.SemaphoreType.DMA,
        pltpu.SemaphoreType.DMA,
        pltpu.SemaphoreType.DMA,
    ]
    if want_counts:
        out_type.append(jax.ShapeDtypeStruct((NC * NS, ACC_ROWS), jnp.float32))
        scratch.append(pltpu.VMEM((ACC_ROWS,), jnp.float32))

    def body(x_hbm, src_hbm, dst_hbm, *rest):
        if want_counts:
            (p_hbm, cnt_hbm, srcb, dstb, rows0, rows1, acc,
             sem0, sem1, semi, cntloc) = rest
        else:
            (p_hbm, srcb, dstb, rows0, rows1, acc,
             sem0, sem1, semi) = rest
        c = lax.axis_index("c")
        s = lax.axis_index("s")
        tile = c * NS + s
        base = tile * CH

        z16 = jnp.zeros((16,), jnp.float32)

        pltpu.async_copy(src_hbm.at[pl.ds(base, W)], srcb.at[0], semi)
        pltpu.async_copy(dst_hbm.at[pl.ds(base, W)], dstb.at[0], semi)

        @pl.loop(0, C)
        def _(r):
            @pl.loop(0, D // 16)
            def _(j):
                rows1[r, pl.ds(j * 16, 16)] = z16

        if want_counts:
            @pl.loop(0, ACC_ROWS // 16)
            def _(i):
                cntloc[pl.ds(i * 16, 16)] = z16

        @pl.loop(0, ROWS_PER_SUB // C)
        def _(k):
            pltpu.sync_copy(rows1, acc.at[pl.ds(s * ROWS_PER_SUB + k * C, C)])

        pltpu.make_async_copy(src_hbm.at[pl.ds(base, W)], srcb.at[0],
                              semi).wait()
        pltpu.make_async_copy(dst_hbm.at[pl.ds(base, W)], dstb.at[0],
                              semi).wait()
        pltpu.async_copy(src_hbm.at[pl.ds(base + W, W)], srcb.at[1], semi)
        pltpu.async_copy(dst_hbm.at[pl.ds(base + W, W)], dstb.at[1], semi)
        pltpu.async_copy(x_hbm.at[srcb.at[0, 0]], rows0, sem0)

        plsc.subcore_barrier()

        ones16 = jnp.full((16,), 1.0, jnp.float32)

        def hist(slot, a):
            if want_counts:
                @pl.loop(0, C // 16)
                def _(j):
                    idx16 = dstb[slot, a, pl.ds(j * 16, 16)]
                    plsc.addupdate_scatter(cntloc, [idx16], ones16)

        @pl.loop(0, NW)
        def _(w):
            slot = lax.rem(w, 2)

            @pl.loop(0, W // 2)
            def _(k):
                a = 2 * k
                last = k == W // 2 - 1
                pltpu.make_async_copy(x_hbm.at[srcb.at[slot, a]], rows0,
                                      sem0).wait()
                pltpu.async_copy(x_hbm.at[srcb.at[slot, a + 1]], rows1, sem1)
                hist(slot, a)
                pltpu.sync_copy(rows0, acc.at[dstb.at[slot, a]], add=True)
                pltpu.make_async_copy(x_hbm.at[srcb.at[slot, a + 1]], rows1,
                                      sem1).wait()

                @pl.when(jnp.logical_not(last))
                def _():
                    pltpu.async_copy(x_hbm.at[srcb.at[slot, a + 2]], rows0,
                                     sem0)

                @pl.when(jnp.logical_and(last, w < NW - 1))
                def _():
                    pltpu.make_async_copy(
                        src_hbm.at[pl.ds(base + (w + 1) * W, W)],
                        srcb.at[1 - slot], semi).wait()
                    pltpu.make_async_copy(
                        dst_hbm.at[pl.ds(base + (w + 1) * W, W)],
                        dstb.at[1 - slot], semi).wait()
                    pltpu.async_copy(x_hbm.at[srcb.at[1 - slot, 0]], rows0,
                                     sem0)

                hist(slot, a + 1)
                pltpu.sync_copy(rows1, acc.at[dstb.at[slot, a + 1]], add=True)

                @pl.when(jnp.logical_and(last, w < NW - 2))
                def _():
                    pltpu.async_copy(src_hbm.at[pl.ds(base + (w + 2) * W, W)],
                                     srcb.at[slot], semi)
                    pltpu.async_copy(dst_hbm.at[pl.ds(base + (w + 2) * W, W)],
                                     dstb.at[slot], semi)

        plsc.subcore_barrier()

        @pl.loop(0, ROWS_PER_SUB // C)
        def _(k):
            r0 = s * ROWS_PER_SUB + k * C
            pltpu.sync_copy(acc.at[pl.ds(r0, C)], p_hbm.at[c, pl.ds(r0, C)])
        if want_counts:
            pltpu.sync_copy(cntloc, cnt_hbm.at[tile])

    cp = pltpu.CompilerParams()
    if "needs_layout_passes" in pltpu.CompilerParams.__dataclass_fields__:
        cp = dataclasses.replace(cp, needs_layout_passes=False)
    fn = pl.kernel(body, mesh=mesh, out_type=tuple(out_type),
                   scratch_types=tuple(scratch), compiler_params=cp)
    return fn(x, src, dst)


def _invcnt_body(cnt_ref, o_ref):
    ones = jnp.ones((NC * NS, 1), jnp.float32)
    col = lax.dot_general(cnt_ref[...], ones, (((0,), (0,)), ((), ())),
                          preferred_element_type=jnp.float32)
    o_ref[...] = 1.0 / jnp.maximum(col, 1.0)


def _tc_invcnt(cnt):
    return pl.pallas_call(
        _invcnt_body,
        out_shape=jax.ShapeDtypeStruct((ACC_ROWS, 1), jnp.float32),
    )(cnt)


def _layer1_body(p_ref, inv_ref, x_ref, wl_ref, b_ref, wr_ref, o_ref):
    agg = (p_ref[0] + p_ref[1]) * inv_ref[...]
    h = (jnp.dot(agg, wl_ref[...], preferred_element_type=jnp.float32)
         + b_ref[...]
         + jnp.dot(x_ref[...], wr_ref[...], preferred_element_type=jnp.float32))
    o_ref[...] = jnp.maximum(h, 0.0)


def _tc_layer1(p, inv, x, Wl, b, Wr):
    return pl.pallas_call(
        _layer1_body,
        grid=(N // BLK,),
        in_specs=[
            pl.BlockSpec((NC, BLK, D), lambda i: (0, i, 0)),
            pl.BlockSpec((BLK, 1), lambda i: (i, 0)),
            pl.BlockSpec((BLK, D), lambda i: (i, 0)),
            pl.BlockSpec((D, D), lambda i: (0, 0)),
            pl.BlockSpec((1, D), lambda i: (0, 0)),
            pl.BlockSpec((D, D), lambda i: (0, 0)),
        ],
        out_specs=pl.BlockSpec((BLK, D), lambda i: (i, 0)),
        out_shape=jax.ShapeDtypeStruct((N, D), jnp.float32),
        compiler_params=pltpu.CompilerParams(
            dimension_semantics=("parallel",)),
    )(p, inv, x, Wl, b, Wr)


def _layer2_body(p_ref, inv_ref, h1_ref, wl_ref, b_ref, wr_ref, batch_ref,
                 wf_ref, bf_ref, o_ref, pool_acc, cntp_acc):
    i = pl.program_id(0)

    @pl.when(i == 0)
    def _():
        pool_acc[...] = jnp.zeros_like(pool_acc)
        cntp_acc[...] = jnp.zeros_like(cntp_acc)

    agg = (p_ref[0] + p_ref[1]) * inv_ref[...]
    h2 = (jnp.dot(agg, wl_ref[...], preferred_element_type=jnp.float32)
          + b_ref[...]
          + jnp.dot(h1_ref[...], wr_ref[...],
                    preferred_element_type=jnp.float32))
    h2 = jnp.maximum(h2, 0.0)

    ids = batch_ref[0, 0, :]
    mask = (lax.broadcasted_iota(jnp.int32, (G, BLK), 0)
            == ids[None, :]).astype(jnp.float32)
    pool_acc[...] += jnp.dot(mask, h2, preferred_element_type=jnp.float32)
    cntp_acc[...] += jnp.sum(mask, axis=1, keepdims=True)

    @pl.when(i == N // BLK - 1)
    def _():
        pooled = pool_acc[...] / jnp.maximum(cntp_acc[...], 1.0)
        logits = (jnp.dot(pooled, wf_ref[...],
                          preferred_element_type=jnp.float32) + bf_ref[...])
        m = jnp.max(logits, axis=-1, keepdims=True)
        lse = jnp.log(jnp.sum(jnp.exp(logits - m), axis=-1, keepdims=True)) + m
        o_ref[...] = logits - lse


def _tc_layer2(p, inv, h1, Wl, b, Wr, batch3, Wf, bf):
    return pl.pallas_call(
        _layer2_body,
        grid=(N // BLK,),
        in_specs=[
            pl.BlockSpec((NC, BLK, D), lambda i: (0, i, 0)),
            pl.BlockSpec((BLK, 1), lambda i: (i, 0)),
            pl.BlockSpec((BLK, D), lambda i: (i, 0)),
            pl.BlockSpec((D, D), lambda i: (0, 0)),
            pl.BlockSpec((1, D), lambda i: (0, 0)),
            pl.BlockSpec((D, D), lambda i: (0, 0)),
            pl.BlockSpec((1, 1, BLK), lambda i: (i, 0, 0)),
            pl.BlockSpec((D, D_OUT), lambda i: (0, 0)),
            pl.BlockSpec((1, D_OUT), lambda i: (0, 0)),
        ],
        out_specs=pl.BlockSpec((G, D_OUT), lambda i: (0, 0)),
        out_shape=jax.ShapeDtypeStruct((G, D_OUT), jnp.float32),
        scratch_shapes=[
            pltpu.VMEM((G, D), jnp.float32),
            pltpu.VMEM((G, D), jnp.float32),
        ],
        compiler_params=pltpu.CompilerParams(
            dimension_semantics=("arbitrary",)),
    )(p, inv, h1, Wl, b, Wr, batch3, Wf, bf)


def kernel(x, edge_index, batch, W1l, b1l, W1r, W2l, b2l, W2r, Wf, bf):
    src = edge_index[0].astype(jnp.int32)
    dst = edge_index[1].astype(jnp.int32)
    pad = EPAD - E
    pad_iota = jnp.arange(pad, dtype=jnp.int32)
    src_p = jnp.concatenate([src, pad_iota % N])
    src_p = src_p.reshape(NC * NS * CH, C)
    dst_p = jnp.concatenate([dst, N + pad_iota % (ACC_ROWS - N)])
    dst_p = dst_p.reshape(NC * NS * CH, C)
    batch3 = batch.astype(jnp.int32).reshape(N // BLK, 1, BLK)
    b1 = b1l.reshape(1, D)
    b2 = b2l.reshape(1, D)
    bfr = bf.reshape(1, D_OUT)

    p1, cnt = _sc_aggregate(x, src_p, dst_p, want_counts=True)
    inv = _tc_invcnt(cnt)
    h1 = _tc_layer1(p1, inv, x, W1l, b1, W1r)
    (p2,) = _sc_aggregate(h1, src_p, dst_p, want_counts=False)
    return _tc_layer2(p2, inv, h1, W2l, b2, W2r, batch3, Wf, bfr)


# speedup vs baseline: 3.6860x; 1.4392x over previous
import dataclasses
import functools

import jax
import jax.numpy as jnp
from jax import lax
from jax.experimental import pallas as pl
from jax.experimental.pallas import tpu as pltpu
from jax.experimental.pallas import tpu_sc as plsc

N = 10000
E = 320000
G = 64
D = 128
D_OUT = 40

NC = 2
NS = 16
C = 64
CH = 160
W = 16
NW = CH // W
EPT = CH * C
EPAD = NC * NS * EPT
ACC_ROWS = 10240
ROWS_PER_SUB = ACC_ROWS // NS

BLK = 1000


def _sc_aggregate(x, src, dst, want_counts, B):
    mesh = plsc.VectorSubcoreMesh(core_axis_name="c", subcore_axis_name="s")
    out_type = [jax.ShapeDtypeStruct((NC, ACC_ROWS, D), jnp.float32)]
    scratch = [
        pltpu.VMEM((2, W, C), jnp.int32),
        pltpu.VMEM((2, W, C), jnp.int32),
        pltpu.VMEM((B, C, D), jnp.float32),
        pltpu.VMEM_SHARED((ACC_ROWS, D), jnp.float32),
        pltpu.SemaphoreType.DMA((B,)),
        pltpu.SemaphoreType.DMA,
    ]
    if want_counts:
        out_type.append(jax.ShapeDtypeStruct((NC * NS, ACC_ROWS), jnp.float32))
        scratch.append(pltpu.VMEM((ACC_ROWS,), jnp.float32))

    def body(x_hbm, src_hbm, dst_hbm, *rest):
        if want_counts:
            (p_hbm, cnt_hbm, srcb, dstb, rows, acc,
             semg, semi, cntloc) = rest
        else:
            (p_hbm, srcb, dstb, rows, acc, semg, semi) = rest
        c = lax.axis_index("c")
        s = lax.axis_index("s")
        tile = c * NS + s
        base = tile * CH

        z16 = jnp.zeros((16,), jnp.float32)

        pltpu.async_copy(src_hbm.at[pl.ds(base, W)], srcb.at[0], semi)
        pltpu.async_copy(dst_hbm.at[pl.ds(base, W)], dstb.at[0], semi)

        @pl.loop(0, C)
        def _(r):
            @pl.loop(0, D // 16)
            def _(j):
                rows[0, r, pl.ds(j * 16, 16)] = z16

        if want_counts:
            @pl.loop(0, ACC_ROWS // 16)
            def _(i):
                cntloc[pl.ds(i * 16, 16)] = z16

        @pl.loop(0, ROWS_PER_SUB // C)
        def _(k):
            pltpu.sync_copy(rows.at[0],
                            acc.at[pl.ds(s * ROWS_PER_SUB + k * C, C)])

        pltpu.make_async_copy(src_hbm.at[pl.ds(base, W)], srcb.at[0],
                              semi).wait()
        pltpu.make_async_copy(dst_hbm.at[pl.ds(base, W)], dstb.at[0],
                              semi).wait()
        pltpu.async_copy(src_hbm.at[pl.ds(base + W, W)], srcb.at[1], semi)
        pltpu.async_copy(dst_hbm.at[pl.ds(base + W, W)], dstb.at[1], semi)
        for b in range(B):
            pltpu.async_copy(x_hbm.at[srcb.at[0, b]], rows.at[b],
                             semg.at[b])

        plsc.subcore_barrier()

        ones16 = jnp.full((16,), 1.0, jnp.float32)

        def hist(slot, a):
            if want_counts:
                for j in range(C // 16):
                    idx16 = dstb[slot, a, pl.ds(j * 16, 16)]
                    plsc.addupdate_scatter(cntloc, [idx16], ones16)

        @pl.loop(0, NW)
        def _(w):
            slot = lax.rem(w, 2)
            nslot = 1 - slot
            for j in range(W):
                b = j % B
                pltpu.make_async_copy(x_hbm.at[srcb.at[slot, j]], rows.at[b],
                                      semg.at[b]).wait()
                hist(slot, j)
                if j == W - B:
                    @pl.when(w < NW - 1)
                    def _():
                        pltpu.make_async_copy(
                            src_hbm.at[pl.ds(base + (w + 1) * W, W)],
                            srcb.at[nslot], semi).wait()
                        pltpu.make_async_copy(
                            dst_hbm.at[pl.ds(base + (w + 1) * W, W)],
                            dstb.at[nslot], semi).wait()
                pltpu.sync_copy(rows.at[b], acc.at[dstb.at[slot, j]],
                                add=True)
                if j < W - B:
                    pltpu.async_copy(x_hbm.at[srcb.at[slot, j + B]],
                                     rows.at[b], semg.at[b])
                else:
                    @pl.when(w < NW - 1)
                    def _():
                        pltpu.async_copy(
                            x_hbm.at[srcb.at[nslot, j - (W - B)]],
                            rows.at[b], semg.at[b])

            @pl.when(w < NW - 2)
            def _():
                pltpu.async_copy(src_hbm.at[pl.ds(base + (w + 2) * W, W)],
                                 srcb.at[slot], semi)
                pltpu.async_copy(dst_hbm.at[pl.ds(base + (w + 2) * W, W)],
                                 dstb.at[slot], semi)

        plsc.subcore_barrier()

        @pl.loop(0, ROWS_PER_SUB // C)
        def _(k):
            r0 = s * ROWS_PER_SUB + k * C
            pltpu.sync_copy(acc.at[pl.ds(r0, C)], p_hbm.at[c, pl.ds(r0, C)])
        if want_counts:
            pltpu.sync_copy(cntloc, cnt_hbm.at[tile])

    cp = pltpu.CompilerParams()
    if "needs_layout_passes" in pltpu.CompilerParams.__dataclass_fields__:
        cp = dataclasses.replace(cp, needs_layout_passes=False)
    fn = pl.kernel(body, mesh=mesh, out_type=tuple(out_type),
                   scratch_types=tuple(scratch), compiler_params=cp)
    return fn(x, src, dst)


def _invcnt_body(cnt_ref, o_ref):
    ones = jnp.ones((NC * NS, 1), jnp.float32)
    col = lax.dot_general(cnt_ref[...], ones, (((0,), (0,)), ((), ())),
                          preferred_element_type=jnp.float32)
    o_ref[...] = 1.0 / jnp.maximum(col, 1.0)


def _tc_invcnt(cnt):
    return pl.pallas_call(
        _invcnt_body,
        out_shape=jax.ShapeDtypeStruct((ACC_ROWS, 1), jnp.float32),
    )(cnt)


def _layer1_body(p_ref, inv_ref, x_ref, wl_ref, b_ref, wr_ref, o_ref):
    agg = (p_ref[0] + p_ref[1]) * inv_ref[...]
    h = (jnp.dot(agg, wl_ref[...], preferred_element_type=jnp.float32)
         + b_ref[...]
         + jnp.dot(x_ref[...], wr_ref[...], preferred_element_type=jnp.float32))
    o_ref[...] = jnp.maximum(h, 0.0)


def _tc_layer1(p, inv, x, Wl, b, Wr):
    return pl.pallas_call(
        _layer1_body,
        grid=(N // BLK,),
        in_specs=[
            pl.BlockSpec((NC, BLK, D), lambda i: (0, i, 0)),
            pl.BlockSpec((BLK, 1), lambda i: (i, 0)),
            pl.BlockSpec((BLK, D), lambda i: (i, 0)),
            pl.BlockSpec((D, D), lambda i: (0, 0)),
            pl.BlockSpec((1, D), lambda i: (0, 0)),
            pl.BlockSpec((D, D), lambda i: (0, 0)),
        ],
        out_specs=pl.BlockSpec((BLK, D), lambda i: (i, 0)),
        out_shape=jax.ShapeDtypeStruct((N, D), jnp.float32),
        compiler_params=pltpu.CompilerParams(
            dimension_semantics=("parallel",)),
    )(p, inv, x, Wl, b, Wr)


def _layer2_body(p_ref, inv_ref, h1_ref, wl_ref, b_ref, wr_ref, batch_ref,
                 wf_ref, bf_ref, o_ref, pool_acc, cntp_acc):
    i = pl.program_id(0)

    @pl.when(i == 0)
    def _():
        pool_acc[...] = jnp.zeros_like(pool_acc)
        cntp_acc[...] = jnp.zeros_like(cntp_acc)

    agg = (p_ref[0] + p_ref[1]) * inv_ref[...]
    h2 = (jnp.dot(agg, wl_ref[...], preferred_element_type=jnp.float32)
          + b_ref[...]
          + jnp.dot(h1_ref[...], wr_ref[...],
                    preferred_element_type=jnp.float32))
    h2 = jnp.maximum(h2, 0.0)

    ids = batch_ref[0, 0, :]
    mask = (lax.broadcasted_iota(jnp.int32, (G, BLK), 0)
            == ids[None, :]).astype(jnp.float32)
    pool_acc[...] += jnp.dot(mask, h2, preferred_element_type=jnp.float32)
    cntp_acc[...] += jnp.sum(mask, axis=1, keepdims=True)

    @pl.when(i == N // BLK - 1)
    def _():
        pooled = pool_acc[...] / jnp.maximum(cntp_acc[...], 1.0)
        logits = (jnp.dot(pooled, wf_ref[...],
                          preferred_element_type=jnp.float32) + bf_ref[...])
        m = jnp.max(logits, axis=-1, keepdims=True)
        lse = jnp.log(jnp.sum(jnp.exp(logits - m), axis=-1, keepdims=True)) + m
        o_ref[...] = logits - lse


def _tc_layer2(p, inv, h1, Wl, b, Wr, batch3, Wf, bf):
    return pl.pallas_call(
        _layer2_body,
        grid=(N // BLK,),
        in_specs=[
            pl.BlockSpec((NC, BLK, D), lambda i: (0, i, 0)),
            pl.BlockSpec((BLK, 1), lambda i: (i, 0)),
            pl.BlockSpec((BLK, D), lambda i: (i, 0)),
            pl.BlockSpec((D, D), lambda i: (0, 0)),
            pl.BlockSpec((1, D), lambda i: (0, 0)),
            pl.BlockSpec((D, D), lambda i: (0, 0)),
            pl.BlockSpec((1, 1, BLK), lambda i: (i, 0, 0)),
            pl.BlockSpec((D, D_OUT), lambda i: (0, 0)),
            pl.BlockSpec((1, D_OUT), lambda i: (0, 0)),
        ],
        out_specs=pl.BlockSpec((G, D_OUT), lambda i: (0, 0)),
        out_shape=jax.ShapeDtypeStruct((G, D_OUT), jnp.float32),
        scratch_shapes=[
            pltpu.VMEM((G, D), jnp.float32),
            pltpu.VMEM((G, D), jnp.float32),
        ],
        compiler_params=pltpu.CompilerParams(
            dimension_semantics=("arbitrary",)),
    )(p, inv, h1, Wl, b, Wr, batch3, Wf, bf)


def kernel(x, edge_index, batch, W1l, b1l, W1r, W2l, b2l, W2r, Wf, bf):
    src = edge_index[0].astype(jnp.int32)
    dst = edge_index[1].astype(jnp.int32)
    pad = EPAD - E
    pad_iota = jnp.arange(pad, dtype=jnp.int32)
    src_p = jnp.concatenate([src, pad_iota % N])
    src_p = src_p.reshape(NC * NS * CH, C)
    dst_p = jnp.concatenate([dst, N + pad_iota % (ACC_ROWS - N)])
    dst_p = dst_p.reshape(NC * NS * CH, C)
    batch3 = batch.astype(jnp.int32).reshape(N // BLK, 1, BLK)
    b1 = b1l.reshape(1, D)
    b2 = b2l.reshape(1, D)
    bfr = bf.reshape(1, D_OUT)

    p1, cnt = _sc_aggregate(x, src_p, dst_p, want_counts=True, B=2)
    inv = _tc_invcnt(cnt)
    h1 = _tc_layer1(p1, inv, x, W1l, b1, W1r)
    (p2,) = _sc_aggregate(h1, src_p, dst_p, want_counts=False, B=4)
    return _tc_layer2(p2, inv, h1, W2l, b2, W2r, batch3, Wf, bfr)
